# baseline (speedup 1.0000x reference)
import dataclasses
import functools

import jax
import jax.numpy as jnp
from jax import lax
from jax.experimental import pallas as pl
from jax.experimental.pallas import tpu as pltpu
from jax.experimental.pallas import tpu_sc as plsc

NU = 25000
NI = 25000
EMB = 64
NS = 16
NC = 2
LANES = 16
KW = 128
NTAB = 25000
RPT = 1568
KW2 = 64
CH = 16
WI = 32
BATCH = 4096
BK = BATCH // (NC * NS)

MODEL_CAT_RATE = 0.02
USER_CAT_RATE = 2.8
ITEM_CAT_RATE = 0.005

_f32 = jnp.float32
_i32 = jnp.int32


def _sc_params():
    cp = pltpu.CompilerParams()
    for field, val in (("needs_layout_passes", False),
                       ("use_tc_tiling_on_sc", False)):
        if field in pltpu.CompilerParams.__dataclass_fields__:
            cp = dataclasses.replace(cp, **{field: val})
    return cp


def _spmm_pair(rows2d, cols2d, vals2d, table0, o0, table1, o1):
    nwin_tot = rows2d.shape[0]
    wpt = nwin_tot // NS
    mesh = plsc.VectorSubcoreMesh(core_axis_name="c", subcore_axis_name="s")
    nbuf = 4

    @functools.partial(
        pl.kernel,
        mesh=mesh,
        out_type=[jax.ShapeDtypeStruct((NTAB, EMB), _f32),
                  jax.ShapeDtypeStruct((NTAB, EMB), _f32)],
        scratch_types=[
            pltpu.VMEM_SHARED((NTAB, EMB), _f32),
            pltpu.VMEM((CH, KW2), _i32),
            pltpu.VMEM((CH, KW2), _i32),
            pltpu.VMEM((CH, KW2), _f32),
            pltpu.VMEM((CH, KW2), _i32),
            pltpu.VMEM((CH, KW2), _i32),
            pltpu.VMEM((CH, KW2), _f32),
            pltpu.VMEM((KW2, EMB), _f32),
            pltpu.VMEM((KW2, EMB), _f32),
            pltpu.VMEM((KW2, EMB), _f32),
            pltpu.VMEM((KW2, EMB), _f32),
        ] + [pltpu.SemaphoreType.DMA] * 10,
        compiler_params=_sc_params(),
    )
    def k(rows_hbm, cols_hbm, vals_hbm, t0_hbm, t1_hbm, out0, out1,
          acc, d0, s0, v0, d1, s1, v1, g0, g1, g2, g3,
          gs0, gs1, gs2, gs3, ss0, ss1, ss2, ss3, is0, is1):
        c = lax.axis_index("c")
        s = lax.axis_index("s")
        idx = [(d0, s0, v0, is0), (d1, s1, v1, is1)]
        gb = [(g0, gs0, ss0), (g1, gs1, ss1), (g2, gs2, ss2), (g3, gs3, ss3)]

        def run(table, out, orient):
            dsth, srch = ((rows_hbm, cols_hbm) if orient == "A"
                          else (cols_hbm, rows_hbm))
            tilebase = s * wpt
            rstart = s * RPT

            @pl.loop(0, KW2)
            def _(r):
                for j in range(EMB // LANES):
                    g0[r, pl.ds(j * LANES, LANES)] = jnp.zeros((LANES,), _f32)

            def zero_rows(nrows):
                @pl.loop(0, nrows // KW2)
                def _(z):
                    pltpu.sync_copy(g0, acc.at[pl.ds(rstart + z * KW2, KW2)])
                tail = nrows - (nrows // KW2) * KW2
                if tail:
                    pltpu.sync_copy(
                        g0.at[pl.ds(0, tail)],
                        acc.at[pl.ds(rstart + (nrows // KW2) * KW2, tail)])

            @pl.when(s < NS - 1)
            def _():
                zero_rows(RPT)

            @pl.when(s == NS - 1)
            def _():
                zero_rows(NTAB - (NS - 1) * RPT)
            plsc.subcore_barrier()

            def idx_load(seti, off, sync=False):
                d, sr, v, sem = idx[seti]
                src_slices = (dsth.at[pl.ds(off, CH)], srch.at[pl.ds(off, CH)],
                              vals_hbm.at[pl.ds(off, CH)])
                for hs, dst in zip(src_slices, (d, sr, v)):
                    if sync:
                        pltpu.sync_copy(hs, dst)
                    else:
                        pltpu.async_copy(hs, dst, sem)

            def idx_wait(seti):
                d, sr, v, sem = idx[seti]
                pltpu.make_async_copy(dsth.at[pl.ds(tilebase, CH)], d, sem).wait()
                pltpu.make_async_copy(srch.at[pl.ds(tilebase, CH)], sr, sem).wait()
                pltpu.make_async_copy(vals_hbm.at[pl.ds(tilebase, CH)], v, sem).wait()

            def gather_start(j):
                seti, jj = divmod(j % WI, CH)
                g, gsem, _ = gb[j % nbuf]
                pltpu.async_copy(table.at[idx[seti][1].at[jj]], g, gsem)

            def gather_wait(j):
                seti, jj = divmod(j % WI, CH)
                g, gsem, _ = gb[j % nbuf]
                pltpu.make_async_copy(table.at[idx[seti][1].at[jj]], g, gsem).wait()

            def scatter_start(j):
                seti, jj = divmod(j % WI, CH)
                g, _, ssem = gb[j % nbuf]
                pltpu.async_copy(g, acc.at[idx[seti][0].at[jj]], ssem, add=True)

            def scatter_wait(j):
                seti, jj = divmod(j % WI, CH)
                g, _, ssem = gb[j % nbuf]
                pltpu.make_async_copy(g, acc.at[idx[seti][0].at[jj]], ssem).wait()

            def scale(j):
                seti, jj = divmod(j % WI, CH)
                g = gb[j % nbuf][0]
                v = idx[seti][2]
                jb = jnp.zeros((LANES,), _i32) + jj

                @pl.loop(0, KW2, unroll=4)
                def _(e):
                    eb = jnp.zeros((LANES,), _i32) + e
                    vb = plsc.load_gather(v, [jb, eb])
                    for j2 in range(EMB // LANES):
                        sl = (e, pl.ds(j2 * LANES, LANES))
                        g[sl] = g[sl] * vb

            idx_load(0, tilebase, sync=True)
            gather_start(0)
            gather_start(1)

            @pl.loop(0, wpt, step=WI)
            def _(w0):
                for j in range(WI):
                    if j < 2:
                        @pl.when(w0 > 0)
                        def _(j=j):
                            scatter_wait(j - 2)
                    else:
                        scatter_wait(j - 2)
                    if j == 2:
                        idx_load(1, tilebase + w0 + CH)
                    if j == CH + 2:
                        @pl.when(w0 + WI < wpt)
                        def _():
                            idx_load(0, tilebase + w0 + WI)
                    if j == CH - 2:
                        idx_wait(1)
                    if j < WI - 2:
                        gather_start(j + 2)
                    else:
                        @pl.when(w0 + WI < wpt)
                        def _(j=j):
                            if j == WI - 2:
                                idx_wait(0)
                            gather_start(j + 2)
                    gather_wait(j)
                    scale(j)
                    scatter_start(j)

            scatter_wait(WI - 2)
            scatter_wait(WI - 1)
            plsc.subcore_barrier()

            @pl.when(s < NS - 1)
            def _():
                pltpu.sync_copy(acc.at[pl.ds(rstart, RPT)],
                                out.at[pl.ds(rstart, RPT)])

            @pl.when(s == NS - 1)
            def _():
                last = NTAB - (NS - 1) * RPT
                pltpu.sync_copy(acc.at[pl.ds(rstart, last)],
                                out.at[pl.ds(rstart, last)])

        @pl.when(c == 0)
        def _():
            run(t0_hbm, out0, o0)

        @pl.when(c == 1)
        def _():
            run(t1_hbm, out1, o1)

    return k(rows2d, cols2d, vals2d, table0, table1)


def _gather21(tables, pairs, idx_u, idx_p, idx_n):
    mesh = plsc.VectorSubcoreMesh(core_axis_name="c", subcore_axis_name="s")
    nt = len(tables)

    @functools.partial(
        pl.kernel,
        mesh=mesh,
        out_type=[jax.ShapeDtypeStruct((BATCH, EMB), _f32)] * len(pairs),
        scratch_types=[pltpu.VMEM((BK,), _i32)] * 3
        + [pltpu.VMEM((BK, EMB), _f32)],
        compiler_params=_sc_params(),
    )
    def k(*refs):
        tabs = refs[:nt]
        idxs = refs[nt:nt + 3]
        outs = refs[nt + 3:nt + 3 + len(pairs)]
        iv = refs[nt + 3 + len(pairs):nt + 6 + len(pairs)]
        gbuf = refs[-1]
        c = lax.axis_index("c")
        s = lax.axis_index("s")
        base = (c * NS + s) * BK
        for j in range(3):
            pltpu.sync_copy(idxs[j].at[pl.ds(base, BK)], iv[j])
        for o, (ti, ii) in zip(outs, pairs):
            pltpu.sync_copy(tabs[ti].at[iv[ii]], gbuf)
            pltpu.sync_copy(gbuf, o.at[pl.ds(base, BK)])

    return k(*tables, idx_u, idx_p, idx_n)


def _mm_bn_stats(x, w, b, gamma, beta):
    nr, d = x.shape
    br = 1000
    nb = nr // br

    def body(x_ref, w_ref, b_ref, g_ref, be_ref, y_ref, st_ref, acc_ref):
        i = pl.program_id(0)
        y = jnp.dot(x_ref[...], w_ref[...],
                    preferred_element_type=_f32) + b_ref[...]
        y_ref[...] = y

        @pl.when(i == 0)
        def _():
            acc_ref[...] = jnp.zeros_like(acc_ref)

        acc_ref[0:1, :] += jnp.sum(y, axis=0, keepdims=True)
        acc_ref[1:2, :] += jnp.sum(y * y, axis=0, keepdims=True)

        @pl.when(i == nb - 1)
        def _():
            mu = acc_ref[0:1, :] * (1.0 / nr)
            var = acc_ref[1:2, :] * (1.0 / nr) - mu * mu
            a = g_ref[...] * lax.rsqrt(var + 1e-5)
            st_ref[0:1, :] = a
            st_ref[1:2, :] = be_ref[...] - mu * a

    return pl.pallas_call(
        body,
        grid=(nb,),
        in_specs=[
            pl.BlockSpec((br, d), lambda i: (i, 0)),
            pl.BlockSpec((d, EMB), lambda i: (0, 0)),
            pl.BlockSpec((1, EMB), lambda i: (0, 0)),
            pl.BlockSpec((1, EMB), lambda i: (0, 0)),
            pl.BlockSpec((1, EMB), lambda i: (0, 0)),
        ],
        out_specs=[
            pl.BlockSpec((br, EMB), lambda i: (i, 0)),
            pl.BlockSpec((2, EMB), lambda i: (0, 0)),
        ],
        out_shape=[
            jax.ShapeDtypeStruct((nr, EMB), _f32),
            jax.ShapeDtypeStruct((2, EMB), _f32),
        ],
        scratch_shapes=[pltpu.VMEM((2, EMB), _f32)],
    )(x, w, b.reshape(1, EMB), gamma.reshape(1, EMB), beta.reshape(1, EMB))


def _bn_apply4(ys, sts):
    nr = ys[0].shape[0]
    br = 1000
    nb = nr // br

    def body(y0, s0, y1, s1, y2, s2, y3, s3, o0, o1, o2, o3):
        for y, st, o in ((y0, s0, o0), (y1, s1, o1), (y2, s2, o2), (y3, s3, o3)):
            o[...] = y[...] * st[0:1, :] + st[1:2, :]

    in_specs = []
    args = []
    for y, st in zip(ys, sts):
        in_specs += [pl.BlockSpec((br, EMB), lambda i: (i, 0)),
                     pl.BlockSpec((2, EMB), lambda i: (0, 0))]
        args += [y, st]
    return pl.pallas_call(
        body,
        grid=(nb,),
        in_specs=in_specs,
        out_specs=[pl.BlockSpec((br, EMB), lambda i: (i, 0))] * 4,
        out_shape=[jax.ShapeDtypeStruct((nr, EMB), _f32)] * 4,
    )(*args)


def _combine(g):
    br = 512
    nb = BATCH // br

    def body(*refs):
        (e0u, e1u, e2u, uimg, utxt, uprof2, uattr,
         e0p, e1p, e2p, iimg2p, itxt2p, iprofp, iattr2p,
         e0n, e1n, e2n, iimg2n, itxt2n, iprofn, iattr2n, out) = refs

        def l2n(ref):
            x = ref[...]
            n = jnp.sqrt(jnp.sum(x * x, axis=1, keepdims=True))
            return x / jnp.maximum(n, 1e-12)

        third = 1.0 / 3.0
        ue = ((e0u[...] + e1u[...] + e2u[...]) * third
              + MODEL_CAT_RATE * l2n(uimg) + MODEL_CAT_RATE * l2n(utxt)
              + USER_CAT_RATE * l2n(uprof2) + ITEM_CAT_RATE * l2n(uattr))
        ip = ((e0p[...] + e1p[...] + e2p[...]) * third
              + MODEL_CAT_RATE * l2n(iimg2p) + MODEL_CAT_RATE * l2n(itxt2p)
              + USER_CAT_RATE * l2n(iprofp) + ITEM_CAT_RATE * l2n(iattr2p))
        inn = ((e0n[...] + e1n[...] + e2n[...]) * third
               + MODEL_CAT_RATE * l2n(iimg2n) + MODEL_CAT_RATE * l2n(itxt2n)
               + USER_CAT_RATE * l2n(iprofn) + ITEM_CAT_RATE * l2n(iattr2n))
        out[...] = jnp.concatenate([
            ue, ip, inn,
            uimg[...], iimg2p[...], iimg2n[...],
            utxt[...], itxt2p[...], itxt2n[...],
            uprof2[...], iprofp[...], iprofn[...],
        ], axis=1)

    return pl.pallas_call(
        body,
        grid=(nb,),
        in_specs=[pl.BlockSpec((br, EMB), lambda i: (i, 0))] * 21,
        out_specs=pl.BlockSpec((br, 12 * EMB), lambda i: (i, 0)),
        out_shape=jax.ShapeDtypeStruct((BATCH, 12 * EMB), _f32),
    )(*g)


def kernel(user_indices, pos_item_indices, neg_item_indices,
           adj_rows, adj_cols, adj_vals,
           int_rows, int_cols, int_vals,
           E0_weight, image_data, text_data, attr_data, prof_data,
           img_W, img_b, img_gamma, img_beta,
           txt_W, txt_b, txt_gamma, txt_beta,
           attr_W, attr_b, attr_gamma, attr_beta,
           prof_W, prof_b, prof_gamma, prof_beta):
    e = int_rows.shape[0]
    wpt = -(-e // (NS * KW2))
    wpt = -(-wpt // WI) * WI
    epad = NS * KW2 * wpt

    def pad2d(a, dtype):
        a = a.astype(dtype)
        a = jnp.pad(a, (0, epad - e))
        return a.reshape(NS * wpt, KW2)

    rows2d = pad2d(int_rows, _i32)
    cols2d = pad2d(int_cols, _i32)
    vals2d = pad2d(int_vals, _f32)
    idx_u = user_indices.astype(_i32)
    idx_p = pos_item_indices.astype(_i32)
    idx_n = neg_item_indices.astype(_i32)

    e0u = E0_weight[:NU]
    e0i = E0_weight[NU:]

    img_y, img_st = _mm_bn_stats(image_data, img_W, img_b, img_gamma, img_beta)
    txt_y, txt_st = _mm_bn_stats(text_data, txt_W, txt_b, txt_gamma, txt_beta)
    attr_y, attr_st = _mm_bn_stats(attr_data, attr_W, attr_b, attr_gamma, attr_beta)
    prof_y, prof_st = _mm_bn_stats(prof_data, prof_W, prof_b, prof_gamma, prof_beta)
    item_img, item_txt, item_attr, user_prof = _bn_apply4(
        (img_y, txt_y, attr_y, prof_y), (img_st, txt_st, attr_st, prof_st))

    e1u, e1i = _spmm_pair(rows2d, cols2d, vals2d, e0i, "A", e0u, "T")
    e2u, e2i = _spmm_pair(rows2d, cols2d, vals2d, e1i, "A", e1u, "T")
    uimg, utxt = _spmm_pair(rows2d, cols2d, vals2d, item_img, "A", item_txt, "A")
    iimg2, itxt2 = _spmm_pair(rows2d, cols2d, vals2d, uimg, "T", utxt, "T")
    uattr, iprof = _spmm_pair(rows2d, cols2d, vals2d, item_attr, "A", user_prof, "T")
    iattr2, uprof2 = _spmm_pair(rows2d, cols2d, vals2d, uattr, "T", iprof, "A")

    tables = (e0u, e1u, e2u, uimg, utxt, uprof2, uattr,
              e0i, e1i, e2i, iimg2, itxt2, iprof, iattr2)
    pairs = ([(0, 0), (1, 0), (2, 0), (3, 0), (4, 0), (5, 0), (6, 0)]
             + [(7, 1), (8, 1), (9, 1), (10, 1), (11, 1), (12, 1), (13, 1)]
             + [(7, 2), (8, 2), (9, 2), (10, 2), (11, 2), (12, 2), (13, 2)])
    g = _gather21(tables, pairs, idx_u, idx_p, idx_n)

    return _combine(g)


# speedup vs baseline: 1.1666x; 1.1666x over previous
import dataclasses
import functools

import jax
import jax.numpy as jnp
from jax import lax
from jax.experimental import pallas as pl
from jax.experimental.pallas import tpu as pltpu
from jax.experimental.pallas import tpu_sc as plsc

NU = 25000
NI = 25000
EMB = 64
NS = 16
NC = 2
LANES = 16
KW = 128
NTAB = 25000
RPT = 1568
KW2 = 64
CH = 8
WI = 16
SLAB = 64
BATCH = 4096
BK = BATCH // (NC * NS)

MODEL_CAT_RATE = 0.02
USER_CAT_RATE = 2.8
ITEM_CAT_RATE = 0.005

_f32 = jnp.float32
_i32 = jnp.int32


def _sc_params():
    cp = pltpu.CompilerParams()
    for field, val in (("needs_layout_passes", False),
                       ("use_tc_tiling_on_sc", False)):
        if field in pltpu.CompilerParams.__dataclass_fields__:
            cp = dataclasses.replace(cp, **{field: val})
    return cp


def _spmm_pair(d0a, s0a, v0a, table0, d1a, s1a, v1a, table1):
    nwin_tot = d0a.shape[0]
    wpt = nwin_tot // NS
    mesh = plsc.VectorSubcoreMesh(core_axis_name="c", subcore_axis_name="s")

    @functools.partial(
        pl.kernel,
        mesh=mesh,
        out_type=jax.ShapeDtypeStruct((2, NTAB, EMB), _f32),
        scratch_types=[
            pltpu.VMEM_SHARED((NTAB, EMB), _f32),
            pltpu.VMEM((CH, KW2), _i32),
            pltpu.VMEM((CH, KW2), _i32),
            pltpu.VMEM((CH, KW2), _f32),
            pltpu.VMEM((CH, KW2), _i32),
            pltpu.VMEM((CH, KW2), _i32),
            pltpu.VMEM((CH, KW2), _f32),
            pltpu.VMEM((SLAB, EMB), _f32),
            pltpu.VMEM((SLAB, EMB), _f32),
            pltpu.VMEM((KW2, EMB), _f32),
            pltpu.VMEM((KW2, EMB), _f32),
            pltpu.VMEM((KW2, EMB), _f32),
            pltpu.VMEM((KW2, EMB), _f32),
        ] + [pltpu.SemaphoreType.DMA] * 8,
        compiler_params=_sc_params(),
    )
    def k(dsth, srch, valh, tabh, outh,
          acc, d0, s0, v0, d1, s1, v1, sb0, sb1, w0b, w1b, w2b, w3b,
          ls0, ls1, ss0, ss1, ss2, ss3, is0, is1):
        c = lax.axis_index("c")
        s = lax.axis_index("s")
        idx = [(d0, s0, v0, is0), (d1, s1, v1, is1)]
        sbufs = [(sb0, ls0), (sb1, ls1)]
        wbufs = [(w0b, ss0), (w1b, ss1), (w2b, ss2), (w3b, ss3)]
        tilebase = s * wpt
        rstart = s * RPT
        table = tabh.at[c]

        @pl.loop(0, KW2)
        def _(r):
            for j in range(EMB // LANES):
                w0b[r, pl.ds(j * LANES, LANES)] = jnp.zeros((LANES,), _f32)

        def zero_rows(nrows):
            @pl.loop(0, nrows // KW2)
            def _(z):
                pltpu.sync_copy(w0b, acc.at[pl.ds(rstart + z * KW2, KW2)])
            tail = nrows - (nrows // KW2) * KW2
            if tail:
                pltpu.sync_copy(
                    w0b.at[pl.ds(0, tail)],
                    acc.at[pl.ds(rstart + (nrows // KW2) * KW2, tail)])

        @pl.when(s < NS - 1)
        def _():
            zero_rows(RPT)

        @pl.when(s == NS - 1)
        def _():
            zero_rows(NTAB - (NS - 1) * RPT)
        plsc.subcore_barrier()

        def idx_load(seti, off, sync=False):
            d, sr, v, sem = idx[seti]
            src_slices = (dsth.at[c, pl.ds(off, CH)],
                          srch.at[c, pl.ds(off, CH)],
                          valh.at[c, pl.ds(off, CH)])
            for hs, dst in zip(src_slices, (d, sr, v)):
                if sync:
                    pltpu.sync_copy(hs, dst)
                else:
                    pltpu.async_copy(hs, dst, sem)

        def idx_wait(seti):
            d, sr, v, sem = idx[seti]
            pltpu.make_async_copy(dsth.at[c, pl.ds(tilebase, CH)], d, sem).wait()
            pltpu.make_async_copy(srch.at[c, pl.ds(tilebase, CH)], sr, sem).wait()
            pltpu.make_async_copy(valh.at[c, pl.ds(tilebase, CH)], v, sem).wait()

        def sw_scalars(p):
            sm = idx[p][1]
            lo = sm[0, pl.ds(0, LANES)][0]
            hi = sm[CH - 1, pl.ds(KW2 - LANES, LANES)][LANES - 1]
            base = jnp.minimum(jnp.bitwise_and(lo, jnp.int32(-8)),
                               jnp.int32(NTAB - SLAB))
            cond = ((hi - base) < SLAB).astype(_i32)
            return base, cond

        def slab_start(p):
            g, lsem = sbufs[p]
            base, cond = sw_scalars(p)

            @pl.when(cond == 1)
            def _():
                pltpu.async_copy(table.at[pl.ds(base, SLAB)], g, lsem)

        def slab_wait(p):
            g, lsem = sbufs[p]
            _, cond = sw_scalars(p)

            @pl.when(cond == 1)
            def _():
                pltpu.make_async_copy(table.at[pl.ds(0, SLAB)], g, lsem).wait()

        def scatter_start(j):
            seti, jj = divmod(j % WI, CH)
            g, ssem = wbufs[j % 4]
            pltpu.async_copy(g, acc.at[idx[seti][0].at[jj]], ssem, add=True)

        def scatter_wait(j):
            seti, jj = divmod(j % WI, CH)
            g, ssem = wbufs[j % 4]
            pltpu.make_async_copy(g, acc.at[idx[seti][0].at[jj]], ssem).wait()

        def scale(j):
            p, jj = divmod(j % WI, CH)
            sb = sbufs[p][0]
            wbf = wbufs[j % 4][0]
            sm, vr = idx[p][1], idx[p][2]
            base, cond = sw_scalars(p)

            @pl.when(cond == 0)
            def _():
                pltpu.sync_copy(table.at[sm.at[jj]], sb.at[pl.ds(0, KW2)])
            roff = jnp.where(cond == 1, base, 0)
            jb = jnp.zeros((LANES,), _i32) + jj
            iota = lax.iota(_i32, LANES)

            @pl.loop(0, KW2, step=LANES)
            def _(e0):
                sv = sm[jj, pl.ds(e0, LANES)]
                sv = jnp.where(cond == 1, sv, iota + e0)
                for i in range(LANES):
                    r = sv[i] - roff
                    eb = jnp.zeros((LANES,), _i32) + (e0 + i)
                    vb = plsc.load_gather(vr, [jb, eb])
                    for kk in range(EMB // LANES):
                        cs = pl.ds(kk * LANES, LANES)
                        wbf[e0 + i, cs] = sb[r, cs] * vb

        idx_load(0, tilebase, sync=True)
        slab_start(0)

        @pl.loop(0, wpt, step=WI)
        def _(w0):
            for j in range(WI):
                if j < 2:
                    @pl.when(w0 > 0)
                    def _(j=j):
                        scatter_wait(j - 2)
                else:
                    scatter_wait(j - 2)
                if j == 2:
                    idx_load(1, tilebase + w0 + CH)
                if j == CH + 2:
                    @pl.when(w0 + WI < wpt)
                    def _():
                        idx_load(0, tilebase + w0 + WI)
                if j == CH - 2:
                    idx_wait(1)
                    slab_start(1)
                if j == WI - 2:
                    @pl.when(w0 + WI < wpt)
                    def _():
                        idx_wait(0)
                        slab_start(0)
                if j == 0:
                    slab_wait(0)
                if j == CH:
                    slab_wait(1)
                scale(j)
                scatter_start(j)

        scatter_wait(WI - 2)
        scatter_wait(WI - 1)
        plsc.subcore_barrier()

        @pl.when(s < NS - 1)
        def _():
            pltpu.sync_copy(acc.at[pl.ds(rstart, RPT)],
                            outh.at[c, pl.ds(rstart, RPT)])

        @pl.when(s == NS - 1)
        def _():
            last = NTAB - (NS - 1) * RPT
            pltpu.sync_copy(acc.at[pl.ds(rstart, last)],
                            outh.at[c, pl.ds(rstart, last)])

    out2 = k(jnp.stack([d0a, d1a]), jnp.stack([s0a, s1a]),
             jnp.stack([v0a, v1a]), jnp.stack([table0, table1]))
    return out2[0], out2[1]


def _gather21(tables, pairs, idx_u, idx_p, idx_n):
    mesh = plsc.VectorSubcoreMesh(core_axis_name="c", subcore_axis_name="s")
    nt = len(tables)

    @functools.partial(
        pl.kernel,
        mesh=mesh,
        out_type=[jax.ShapeDtypeStruct((BATCH, EMB), _f32)] * len(pairs),
        scratch_types=[pltpu.VMEM((BK,), _i32)] * 3
        + [pltpu.VMEM((BK, EMB), _f32)],
        compiler_params=_sc_params(),
    )
    def k(*refs):
        tabs = refs[:nt]
        idxs = refs[nt:nt + 3]
        outs = refs[nt + 3:nt + 3 + len(pairs)]
        iv = refs[nt + 3 + len(pairs):nt + 6 + len(pairs)]
        gbuf = refs[-1]
        c = lax.axis_index("c")
        s = lax.axis_index("s")
        base = (c * NS + s) * BK
        for j in range(3):
            pltpu.sync_copy(idxs[j].at[pl.ds(base, BK)], iv[j])
        for o, (ti, ii) in zip(outs, pairs):
            pltpu.sync_copy(tabs[ti].at[iv[ii]], gbuf)
            pltpu.sync_copy(gbuf, o.at[pl.ds(base, BK)])

    return k(*tables, idx_u, idx_p, idx_n)


def _mm_bn_stats(x, w, b, gamma, beta):
    nr, d = x.shape
    br = 1000
    nb = nr // br

    def body(x_ref, w_ref, b_ref, g_ref, be_ref, y_ref, st_ref, acc_ref):
        i = pl.program_id(0)
        y = jnp.dot(x_ref[...], w_ref[...],
                    preferred_element_type=_f32) + b_ref[...]
        y_ref[...] = y

        @pl.when(i == 0)
        def _():
            acc_ref[...] = jnp.zeros_like(acc_ref)

        acc_ref[0:1, :] += jnp.sum(y, axis=0, keepdims=True)
        acc_ref[1:2, :] += jnp.sum(y * y, axis=0, keepdims=True)

        @pl.when(i == nb - 1)
        def _():
            mu = acc_ref[0:1, :] * (1.0 / nr)
            var = acc_ref[1:2, :] * (1.0 / nr) - mu * mu
            a = g_ref[...] * lax.rsqrt(var + 1e-5)
            st_ref[0:1, :] = a
            st_ref[1:2, :] = be_ref[...] - mu * a

    return pl.pallas_call(
        body,
        grid=(nb,),
        in_specs=[
            pl.BlockSpec((br, d), lambda i: (i, 0)),
            pl.BlockSpec((d, EMB), lambda i: (0, 0)),
            pl.BlockSpec((1, EMB), lambda i: (0, 0)),
            pl.BlockSpec((1, EMB), lambda i: (0, 0)),
            pl.BlockSpec((1, EMB), lambda i: (0, 0)),
        ],
        out_specs=[
            pl.BlockSpec((br, EMB), lambda i: (i, 0)),
            pl.BlockSpec((2, EMB), lambda i: (0, 0)),
        ],
        out_shape=[
            jax.ShapeDtypeStruct((nr, EMB), _f32),
            jax.ShapeDtypeStruct((2, EMB), _f32),
        ],
        scratch_shapes=[pltpu.VMEM((2, EMB), _f32)],
    )(x, w, b.reshape(1, EMB), gamma.reshape(1, EMB), beta.reshape(1, EMB))


def _bn_apply4(ys, sts):
    nr = ys[0].shape[0]
    br = 1000
    nb = nr // br

    def body(y0, s0, y1, s1, y2, s2, y3, s3, o0, o1, o2, o3):
        for y, st, o in ((y0, s0, o0), (y1, s1, o1), (y2, s2, o2), (y3, s3, o3)):
            o[...] = y[...] * st[0:1, :] + st[1:2, :]

    in_specs = []
    args = []
    for y, st in zip(ys, sts):
        in_specs += [pl.BlockSpec((br, EMB), lambda i: (i, 0)),
                     pl.BlockSpec((2, EMB), lambda i: (0, 0))]
        args += [y, st]
    return pl.pallas_call(
        body,
        grid=(nb,),
        in_specs=in_specs,
        out_specs=[pl.BlockSpec((br, EMB), lambda i: (i, 0))] * 4,
        out_shape=[jax.ShapeDtypeStruct((nr, EMB), _f32)] * 4,
    )(*args)


def _combine(g):
    br = 512
    nb = BATCH // br

    def body(*refs):
        (e0u, e1u, e2u, uimg, utxt, uprof2, uattr,
         e0p, e1p, e2p, iimg2p, itxt2p, iprofp, iattr2p,
         e0n, e1n, e2n, iimg2n, itxt2n, iprofn, iattr2n, out) = refs

        def l2n(ref):
            x = ref[...]
            n = jnp.sqrt(jnp.sum(x * x, axis=1, keepdims=True))
            return x / jnp.maximum(n, 1e-12)

        third = 1.0 / 3.0
        ue = ((e0u[...] + e1u[...] + e2u[...]) * third
              + MODEL_CAT_RATE * l2n(uimg) + MODEL_CAT_RATE * l2n(utxt)
              + USER_CAT_RATE * l2n(uprof2) + ITEM_CAT_RATE * l2n(uattr))
        ip = ((e0p[...] + e1p[...] + e2p[...]) * third
              + MODEL_CAT_RATE * l2n(iimg2p) + MODEL_CAT_RATE * l2n(itxt2p)
              + USER_CAT_RATE * l2n(iprofp) + ITEM_CAT_RATE * l2n(iattr2p))
        inn = ((e0n[...] + e1n[...] + e2n[...]) * third
               + MODEL_CAT_RATE * l2n(iimg2n) + MODEL_CAT_RATE * l2n(itxt2n)
               + USER_CAT_RATE * l2n(iprofn) + ITEM_CAT_RATE * l2n(iattr2n))
        out[...] = jnp.concatenate([
            ue, ip, inn,
            uimg[...], iimg2p[...], iimg2n[...],
            utxt[...], itxt2p[...], itxt2n[...],
            uprof2[...], iprofp[...], iprofn[...],
        ], axis=1)

    return pl.pallas_call(
        body,
        grid=(nb,),
        in_specs=[pl.BlockSpec((br, EMB), lambda i: (i, 0))] * 21,
        out_specs=pl.BlockSpec((br, 12 * EMB), lambda i: (i, 0)),
        out_shape=jax.ShapeDtypeStruct((BATCH, 12 * EMB), _f32),
    )(*g)


def kernel(user_indices, pos_item_indices, neg_item_indices,
           adj_rows, adj_cols, adj_vals,
           int_rows, int_cols, int_vals,
           E0_weight, image_data, text_data, attr_data, prof_data,
           img_W, img_b, img_gamma, img_beta,
           txt_W, txt_b, txt_gamma, txt_beta,
           attr_W, attr_b, attr_gamma, attr_beta,
           prof_W, prof_b, prof_gamma, prof_beta):
    e = int_rows.shape[0]
    wpt = -(-e // (NS * KW2))
    wpt = -(-wpt // WI) * WI
    epad = NS * KW2 * wpt

    def pad2d(a, dtype, fill=0):
        a = a.astype(dtype)
        a = jnp.pad(a, (0, epad - e), constant_values=fill)
        return a.reshape(NS * wpt, KW2)

    perm = jnp.argsort(int_cols)
    rows2d = pad2d(int_rows, _i32, NTAB - 1)
    cols2d = pad2d(int_cols, _i32)
    vals2d = pad2d(int_vals, _f32)
    rows2dp = pad2d(int_rows[perm], _i32)
    cols2dp = pad2d(int_cols[perm], _i32, NTAB - 1)
    vals2dp = pad2d(int_vals[perm], _f32)
    A = (rows2dp, cols2dp, vals2dp)
    T = (cols2d, rows2d, vals2d)
    idx_u = user_indices.astype(_i32)
    idx_p = pos_item_indices.astype(_i32)
    idx_n = neg_item_indices.astype(_i32)

    e0u = E0_weight[:NU]
    e0i = E0_weight[NU:]

    img_y, img_st = _mm_bn_stats(image_data, img_W, img_b, img_gamma, img_beta)
    txt_y, txt_st = _mm_bn_stats(text_data, txt_W, txt_b, txt_gamma, txt_beta)
    attr_y, attr_st = _mm_bn_stats(attr_data, attr_W, attr_b, attr_gamma, attr_beta)
    prof_y, prof_st = _mm_bn_stats(prof_data, prof_W, prof_b, prof_gamma, prof_beta)
    item_img, item_txt, item_attr, user_prof = _bn_apply4(
        (img_y, txt_y, attr_y, prof_y), (img_st, txt_st, attr_st, prof_st))

    e1u, e1i = _spmm_pair(*A, e0i, *T, e0u)
    e2u, e2i = _spmm_pair(*A, e1i, *T, e1u)
    uimg, utxt = _spmm_pair(*A, item_img, *A, item_txt)
    iimg2, itxt2 = _spmm_pair(*T, uimg, *T, utxt)
    uattr, iprof = _spmm_pair(*A, item_attr, *T, user_prof)
    iattr2, uprof2 = _spmm_pair(*T, uattr, *A, iprof)

    tables = (e0u, e1u, e2u, uimg, utxt, uprof2, uattr,
              e0i, e1i, e2i, iimg2, itxt2, iprof, iattr2)
    pairs = ([(0, 0), (1, 0), (2, 0), (3, 0), (4, 0), (5, 0), (6, 0)]
             + [(7, 1), (8, 1), (9, 1), (10, 1), (11, 1), (12, 1), (13, 1)]
             + [(7, 2), (8, 2), (9, 2), (10, 2), (11, 2), (12, 2), (13, 2)])
    g = _gather21(tables, pairs, idx_u, idx_p, idx_n)

    return _combine(g)
